# baseline (speedup 1.0000x reference)
import functools

import jax
import jax.numpy as jnp
from jax import lax
from jax.experimental import pallas as pl
from jax.experimental.pallas import tpu as pltpu
from jax.experimental.pallas import tpu_sc as plsc

_N = 10000
_E = 320000
_F = 128
_H = 64
_B = 16
_C = 10

_NW = 32
_CH = 64
_NCH = 160
_E_PAD = _NW * _CH * _NCH
_N_PAD = 10112
_RPT = _N_PAD // 16


def _make_sc_agg(F):
    mesh = plsc.VectorSubcoreMesh(core_axis_name="c", subcore_axis_name="s")

    NB = 2
    NP = 2
    PCH = _NCH // NP

    @functools.partial(
        pl.kernel,
        out_type=jax.ShapeDtypeStruct((2, _N_PAD, F), jnp.float32),
        mesh=mesh,
        scratch_types=[
            pltpu.VMEM((PCH, _CH), jnp.int32),
            pltpu.VMEM((PCH, _CH), jnp.int32),
            pltpu.VMEM((_CH, F), jnp.float32),
            pltpu.VMEM((_CH, F), jnp.float32),
            pltpu.VMEM_SHARED((_N_PAD, F), jnp.float32),
            pltpu.SemaphoreType.DMA,
            pltpu.SemaphoreType.DMA,
        ],
    )
    def agg(src_hbm, dst_hbm, h_hbm, zrow_hbm, out_hbm,
            src_v, dst_v, rows0, rows1, acc, gsem, ssem):
        bufs = (rows0, rows1)
        cid = lax.axis_index("c")
        sid = lax.axis_index("s")
        wid = sid * 2 + cid
        pltpu.sync_copy(zrow_hbm, acc.at[pl.ds(sid * _RPT, _RPT)])
        plsc.subcore_barrier()

        def body(t, carry):
            r = lax.rem(t, NB)
            pltpu.make_async_copy(h_hbm.at[src_v.at[t]], bufs[0], gsem).wait()
            for i in range(NB):

                @pl.when(r == i)
                def _scatter(i=i):
                    pltpu.async_copy(bufs[i], acc.at[dst_v.at[t]],
                                     ssem, add=True)

            @pl.when(t + 1 < PCH)
            def _prefetch():
                @pl.when(t >= 1)
                def _drain():
                    pltpu.make_async_copy(h_hbm.at[src_v.at[t]], bufs[0],
                                          ssem).wait()

                r2 = lax.rem(t + 1, NB)
                for i in range(NB):

                    @pl.when(r2 == i)
                    def _gather(i=i):
                        pltpu.async_copy(h_hbm.at[src_v.at[t + 1]], bufs[i],
                                         gsem)

            return carry

        for p in range(NP):
            pltpu.sync_copy(
                src_hbm.at[pl.ds(wid * _NCH + p * PCH, PCH)], src_v)
            pltpu.sync_copy(
                dst_hbm.at[pl.ds(wid * _NCH + p * PCH, PCH)], dst_v)
            pltpu.async_copy(h_hbm.at[src_v.at[0]], bufs[0], gsem)
            lax.fori_loop(0, PCH, body, 0)
            for _ in range(2):
                pltpu.make_async_copy(h_hbm.at[src_v.at[0]], bufs[0],
                                      ssem).wait()
        plsc.subcore_barrier()
        pltpu.sync_copy(acc.at[pl.ds(sid * _RPT, _RPT)],
                        out_hbm.at[cid, pl.ds(sid * _RPT, _RPT)])

    return agg


_sc_agg_f = _make_sc_agg(_F)


def _bn_relu(z, g, b):
    m = jnp.mean(z, axis=0, keepdims=True)
    v = jnp.mean((z - m) * (z - m), axis=0, keepdims=True)
    return jax.nn.relu((z - m) * lax.rsqrt(v + 1e-5) * g + b)


def _dense0_body(x_ref, agg_ref, w1_ref, b1_ref, g1_ref, be1_ref,
                 w2_ref, b2_ref, bg_ref, bb_ref, h1_ref):
    u = x_ref[...] + agg_ref[0, :_N, :] + agg_ref[1, :_N, :]
    z = jnp.dot(u, w1_ref[...], preferred_element_type=jnp.float32) + b1_ref[...]
    z = _bn_relu(z, g1_ref[...], be1_ref[...])
    z = jnp.dot(z, w2_ref[...], preferred_element_type=jnp.float32) + b2_ref[...]
    h1 = _bn_relu(z, bg_ref[...], bb_ref[...])
    h1_ref[...] = jnp.concatenate([h1, jnp.zeros((_N, _F - _H), jnp.float32)],
                                  axis=1)


def _dense1_body(x_ref, h1_ref, agg_ref, batch_ref, w1_ref, b1_ref, g1_ref,
                 be1_ref, w2_ref, b2_ref, bg_ref, bb_ref,
                 l0w_ref, l0b_ref, l1w_ref, l1b_ref, l2w_ref, l2b_ref,
                 hw_ref, out_ref):
    h1 = h1_ref[:, :_H]
    u = h1 + agg_ref[0, :_N, :_H] + agg_ref[1, :_N, :_H]
    z = jnp.dot(u, w1_ref[...], preferred_element_type=jnp.float32) + b1_ref[...]
    z = _bn_relu(z, g1_ref[...], be1_ref[...])
    z = jnp.dot(z, w2_ref[...], preferred_element_type=jnp.float32) + b2_ref[...]
    h2 = _bn_relu(z, bg_ref[...], bb_ref[...])

    iot = lax.broadcasted_iota(jnp.int32, (_B, _N), 0)
    oh = jnp.where(iot == batch_ref[...], 1.0, 0.0).astype(jnp.float32)
    cnt = jnp.sum(oh, axis=1, keepdims=True)
    scale = 1.0 / jnp.maximum(cnt, 1.0)
    p0 = jnp.dot(oh, x_ref[...], preferred_element_type=jnp.float32) * scale
    p1 = jnp.dot(oh, h1, preferred_element_type=jnp.float32) * scale
    p2 = jnp.dot(oh, h2, preferred_element_type=jnp.float32) * scale
    hw = hw_ref[...]
    o = (jnp.dot(p0, l0w_ref[...], preferred_element_type=jnp.float32)
         + l0b_ref[...]) * hw[:, 0:1]
    o += (jnp.dot(p1, l1w_ref[...], preferred_element_type=jnp.float32)
          + l1b_ref[...]) * hw[:, 1:2]
    o += (jnp.dot(p2, l2w_ref[...], preferred_element_type=jnp.float32)
          + l2b_ref[...]) * hw[:, 2:3]
    out_ref[...] = o


_dense0 = pl.pallas_call(
    _dense0_body,
    out_shape=jax.ShapeDtypeStruct((_N, _F), jnp.float32),
)

_dense1 = pl.pallas_call(
    _dense1_body,
    out_shape=jax.ShapeDtypeStruct((_B, _C), jnp.float32),
)


def kernel(x, edge_index, batch, hop_weights,
           c0_w1, c0_b1, c0_g1, c0_be1, c0_w2, c0_b2,
           c1_w1, c1_b1, c1_g1, c1_be1, c1_w2, c1_b2,
           bn0_g, bn0_b, bn1_g, bn1_b,
           l0_w, l0_b, l1_w, l1_b, l2_w, l2_b):
    pad = _E_PAD - _E
    src = jnp.concatenate([edge_index[0], jnp.zeros((pad,), jnp.int32)])
    dst = jnp.concatenate([edge_index[1], jnp.full((pad,), _N, jnp.int32)])
    src2 = src.reshape(-1, _CH)
    dst2 = dst.reshape(-1, _CH)
    zf = jnp.zeros((_RPT, _F), jnp.float32)

    r = lambda a: a.reshape(1, -1)

    agg0 = _sc_agg_f(src2, dst2, x, zf)
    h1 = _dense0(x, agg0, c0_w1, r(c0_b1), r(c0_g1), r(c0_be1),
                 c0_w2, r(c0_b2), r(bn0_g), r(bn0_b))
    agg1 = _sc_agg_f(src2, dst2, h1, zf)
    out = _dense1(x, h1, agg1, batch.reshape(1, _N), c1_w1, r(c1_b1),
                  r(c1_g1), r(c1_be1), c1_w2, r(c1_b2), r(bn1_g), r(bn1_b),
                  l0_w, r(l0_b), l1_w, r(l1_b), l2_w, r(l2_b), hop_weights)
    return out


# speedup vs baseline: 1.1531x; 1.1531x over previous
import functools

import jax
import jax.numpy as jnp
from jax import lax
from jax.experimental import pallas as pl
from jax.experimental.pallas import tpu as pltpu
from jax.experimental.pallas import tpu_sc as plsc

_N = 10000
_E = 320000
_F = 128
_H = 64
_B = 16
_C = 10

_NW = 32
_CH = 64
_CPT0 = 232
_CPT1 = 88
_NROW = 16 * (_CPT0 + _CPT1)
_E_PAD = _NROW * _CH
_PCH = 80
_NPASS = 3
_ROW_PAD = _NROW + 2 * _PCH
_N_PAD = 10112
_RPT = _N_PAD // 16


def _make_sc_agg(F):
    mesh = plsc.VectorSubcoreMesh(core_axis_name="c", subcore_axis_name="s")

    NB = 2

    @functools.partial(
        pl.kernel,
        out_type=jax.ShapeDtypeStruct((2, _N_PAD, F), jnp.float32),
        mesh=mesh,
        scratch_types=[
            pltpu.VMEM((_PCH, _CH), jnp.int32),
            pltpu.VMEM((_PCH, _CH), jnp.int32),
            pltpu.VMEM((_CH, F), jnp.float32),
            pltpu.VMEM((_CH, F), jnp.float32),
            pltpu.VMEM_SHARED((_N_PAD, F), jnp.float32),
            pltpu.SemaphoreType.DMA,
            pltpu.SemaphoreType.DMA,
        ],
    )
    def agg(src_hbm, dst_hbm, h_hbm, zrow_hbm, out_hbm,
            src_v, dst_v, rows0, rows1, acc, gsem, ssem):
        bufs = (rows0, rows1)
        cid = lax.axis_index("c")
        sid = lax.axis_index("s")
        pltpu.sync_copy(zrow_hbm, acc.at[pl.ds(sid * _RPT, _RPT)])
        plsc.subcore_barrier()

        my_base = jnp.where(cid == 0, sid * _CPT0, 16 * _CPT0 + sid * _CPT1)
        my_n = jnp.where(cid == 0, _CPT0, _CPT1)

        for p in range(_NPASS):
            cnt = jnp.minimum(my_n - p * _PCH, _PCH)

            @pl.when(cnt > 0)
            def _one_pass(p=p, cnt=cnt):
                pltpu.sync_copy(
                    src_hbm.at[pl.ds(my_base + p * _PCH, _PCH)], src_v)
                pltpu.sync_copy(
                    dst_hbm.at[pl.ds(my_base + p * _PCH, _PCH)], dst_v)
                pltpu.async_copy(h_hbm.at[src_v.at[0]], bufs[0], gsem)

                def body(t, carry):
                    r = lax.rem(t, NB)
                    pltpu.make_async_copy(h_hbm.at[src_v.at[t]], bufs[0],
                                          gsem).wait()
                    for i in range(NB):

                        @pl.when(r == i)
                        def _scatter(i=i):
                            pltpu.async_copy(bufs[i], acc.at[dst_v.at[t]],
                                             ssem, add=True)

                    @pl.when(t + 1 < cnt)
                    def _prefetch():
                        @pl.when(t >= 1)
                        def _drain():
                            pltpu.make_async_copy(h_hbm.at[src_v.at[t]],
                                                  bufs[0], ssem).wait()

                        r2 = lax.rem(t + 1, NB)
                        for i in range(NB):

                            @pl.when(r2 == i)
                            def _gather(i=i):
                                pltpu.async_copy(h_hbm.at[src_v.at[t + 1]],
                                                 bufs[i], gsem)

                    return carry

                lax.fori_loop(0, cnt, body, 0)
                for _ in range(2):
                    pltpu.make_async_copy(h_hbm.at[src_v.at[0]], bufs[0],
                                          ssem).wait()

        plsc.subcore_barrier()
        pltpu.sync_copy(acc.at[pl.ds(sid * _RPT, _RPT)],
                        out_hbm.at[cid, pl.ds(sid * _RPT, _RPT)])

    return agg


_sc_agg_f = _make_sc_agg(_F)


def _bn_relu(z, g, b):
    m = jnp.mean(z, axis=0, keepdims=True)
    v = jnp.mean((z - m) * (z - m), axis=0, keepdims=True)
    return jax.nn.relu((z - m) * lax.rsqrt(v + 1e-5) * g + b)


def _dense0_body(x_ref, agg_ref, w1_ref, b1_ref, g1_ref, be1_ref,
                 w2_ref, b2_ref, bg_ref, bb_ref, h1_ref):
    u = x_ref[...] + agg_ref[0, :_N, :] + agg_ref[1, :_N, :]
    z = jnp.dot(u, w1_ref[...], preferred_element_type=jnp.float32) + b1_ref[...]
    z = _bn_relu(z, g1_ref[...], be1_ref[...])
    z = jnp.dot(z, w2_ref[...], preferred_element_type=jnp.float32) + b2_ref[...]
    h1 = _bn_relu(z, bg_ref[...], bb_ref[...])
    h1_ref[...] = jnp.concatenate([h1, jnp.zeros((_N, _F - _H), jnp.float32)],
                                  axis=1)


def _dense1_body(x_ref, h1_ref, agg_ref, batch_ref, w1_ref, b1_ref, g1_ref,
                 be1_ref, w2_ref, b2_ref, bg_ref, bb_ref,
                 l0w_ref, l0b_ref, l1w_ref, l1b_ref, l2w_ref, l2b_ref,
                 hw_ref, out_ref):
    h1 = h1_ref[:, :_H]
    u = h1 + agg_ref[0, :_N, :_H] + agg_ref[1, :_N, :_H]
    z = jnp.dot(u, w1_ref[...], preferred_element_type=jnp.float32) + b1_ref[...]
    z = _bn_relu(z, g1_ref[...], be1_ref[...])
    z = jnp.dot(z, w2_ref[...], preferred_element_type=jnp.float32) + b2_ref[...]
    h2 = _bn_relu(z, bg_ref[...], bb_ref[...])

    iot = lax.broadcasted_iota(jnp.int32, (_B, _N), 0)
    oh = jnp.where(iot == batch_ref[...], 1.0, 0.0).astype(jnp.float32)
    cnt = jnp.sum(oh, axis=1, keepdims=True)
    scale = 1.0 / jnp.maximum(cnt, 1.0)
    p0 = jnp.dot(oh, x_ref[...], preferred_element_type=jnp.float32) * scale
    p1 = jnp.dot(oh, h1, preferred_element_type=jnp.float32) * scale
    p2 = jnp.dot(oh, h2, preferred_element_type=jnp.float32) * scale
    hw = hw_ref[...]
    o = (jnp.dot(p0, l0w_ref[...], preferred_element_type=jnp.float32)
         + l0b_ref[...]) * hw[:, 0:1]
    o += (jnp.dot(p1, l1w_ref[...], preferred_element_type=jnp.float32)
          + l1b_ref[...]) * hw[:, 1:2]
    o += (jnp.dot(p2, l2w_ref[...], preferred_element_type=jnp.float32)
          + l2b_ref[...]) * hw[:, 2:3]
    out_ref[...] = o


_dense0 = pl.pallas_call(
    _dense0_body,
    out_shape=jax.ShapeDtypeStruct((_N, _F), jnp.float32),
)

_dense1 = pl.pallas_call(
    _dense1_body,
    out_shape=jax.ShapeDtypeStruct((_B, _C), jnp.float32),
)


def kernel(x, edge_index, batch, hop_weights,
           c0_w1, c0_b1, c0_g1, c0_be1, c0_w2, c0_b2,
           c1_w1, c1_b1, c1_g1, c1_be1, c1_w2, c1_b2,
           bn0_g, bn0_b, bn1_g, bn1_b,
           l0_w, l0_b, l1_w, l1_b, l2_w, l2_b):
    pad = _ROW_PAD * _CH - _E
    src = jnp.concatenate([edge_index[0], jnp.zeros((pad,), jnp.int32)])
    dst = jnp.concatenate([edge_index[1], jnp.full((pad,), _N, jnp.int32)])
    src2 = src.reshape(-1, _CH)
    dst2 = dst.reshape(-1, _CH)
    zf = jnp.zeros((_RPT, _F), jnp.float32)

    r = lambda a: a.reshape(1, -1)

    agg0 = _sc_agg_f(src2, dst2, x, zf)
    h1 = _dense0(x, agg0, c0_w1, r(c0_b1), r(c0_g1), r(c0_be1),
                 c0_w2, r(c0_b2), r(bn0_g), r(bn0_b))
    agg1 = _sc_agg_f(src2, dst2, h1, zf)
    out = _dense1(x, h1, agg1, batch.reshape(1, _N), c1_w1, r(c1_b1),
                  r(c1_g1), r(c1_be1), c1_w2, r(c1_b2), r(bn1_g), r(bn1_b),
                  l0_w, r(l0_b), l1_w, r(l1_b), l2_w, r(l2_b), hop_weights)
    return out


# speedup vs baseline: 1.1805x; 1.0238x over previous
import functools

import jax
import jax.numpy as jnp
from jax import lax
from jax.experimental import pallas as pl
from jax.experimental.pallas import tpu as pltpu
from jax.experimental.pallas import tpu_sc as plsc

_N = 10000
_E = 320000
_F = 128
_H = 64
_B = 16
_C = 10

_NW = 32
_CH = 32
_CPT0 = 464
_CPT1 = 176
_NROW = 16 * (_CPT0 + _CPT1)
_E_PAD = _NROW * _CH
_PCH = 96
_NPASS = 5
_ROW_PAD = _NROW + 2 * _PCH
_N_PAD = 10112
_RPT = _N_PAD // 16


def _make_sc_agg(F):
    mesh = plsc.VectorSubcoreMesh(core_axis_name="c", subcore_axis_name="s")

    NB = 4

    @functools.partial(
        pl.kernel,
        out_type=jax.ShapeDtypeStruct((2, _N_PAD, F), jnp.float32),
        mesh=mesh,
        scratch_types=[
            pltpu.VMEM((_PCH, _CH), jnp.int32),
            pltpu.VMEM((_PCH, _CH), jnp.int32),
            pltpu.VMEM((_CH, F), jnp.float32),
            pltpu.VMEM((_CH, F), jnp.float32),
            pltpu.VMEM((_CH, F), jnp.float32),
            pltpu.VMEM((_CH, F), jnp.float32),
            pltpu.VMEM_SHARED((_N_PAD, F), jnp.float32),
            pltpu.SemaphoreType.DMA,
            pltpu.SemaphoreType.DMA,
        ],
    )
    def agg(src_hbm, dst_hbm, h_hbm, zrow_hbm, out_hbm,
            src_v, dst_v, rows0, rows1, rows2, rows3, acc, gsem, ssem):
        bufs = (rows0, rows1, rows2, rows3)
        cid = lax.axis_index("c")
        sid = lax.axis_index("s")
        pltpu.sync_copy(zrow_hbm, acc.at[pl.ds(sid * _RPT, _RPT)])
        plsc.subcore_barrier()

        my_base = jnp.where(cid == 0, sid * _CPT0, 16 * _CPT0 + sid * _CPT1)
        my_n = jnp.where(cid == 0, _CPT0, _CPT1)

        for p in range(_NPASS):
            cnt = jnp.minimum(my_n - p * _PCH, _PCH)

            @pl.when(cnt > 0)
            def _one_pass(p=p, cnt=cnt):
                pltpu.sync_copy(
                    src_hbm.at[pl.ds(my_base + p * _PCH, _PCH)], src_v)
                pltpu.sync_copy(
                    dst_hbm.at[pl.ds(my_base + p * _PCH, _PCH)], dst_v)
                pltpu.async_copy(h_hbm.at[src_v.at[0]], bufs[0], gsem)
                pltpu.async_copy(h_hbm.at[src_v.at[1]], bufs[1], gsem)
                pltpu.async_copy(h_hbm.at[src_v.at[2]], bufs[2], gsem)

                def body(t, carry):
                    r = lax.rem(t, NB)
                    pltpu.make_async_copy(h_hbm.at[src_v.at[t]], bufs[0],
                                          gsem).wait()
                    for i in range(NB):

                        @pl.when(r == i)
                        def _scatter(i=i):
                            pltpu.async_copy(bufs[i], acc.at[dst_v.at[t]],
                                             ssem, add=True)

                    @pl.when(t + 3 < cnt)
                    def _prefetch():
                        @pl.when(t >= 1)
                        def _drain():
                            pltpu.make_async_copy(h_hbm.at[src_v.at[t]],
                                                  bufs[0], ssem).wait()

                        r2 = lax.rem(t + 3, NB)
                        for i in range(NB):

                            @pl.when(r2 == i)
                            def _gather(i=i):
                                pltpu.async_copy(h_hbm.at[src_v.at[t + 3]],
                                                 bufs[i], gsem)

                    return carry

                lax.fori_loop(0, cnt, body, 0)
                for _ in range(4):
                    pltpu.make_async_copy(h_hbm.at[src_v.at[0]], bufs[0],
                                          ssem).wait()

        plsc.subcore_barrier()
        pltpu.sync_copy(acc.at[pl.ds(sid * _RPT, _RPT)],
                        out_hbm.at[cid, pl.ds(sid * _RPT, _RPT)])

    return agg


_sc_agg_f = _make_sc_agg(_F)


def _bn_relu(z, g, b):
    m = jnp.mean(z, axis=0, keepdims=True)
    v = jnp.mean((z - m) * (z - m), axis=0, keepdims=True)
    return jax.nn.relu((z - m) * lax.rsqrt(v + 1e-5) * g + b)


def _dense0_body(x_ref, agg_ref, w1_ref, b1_ref, g1_ref, be1_ref,
                 w2_ref, b2_ref, bg_ref, bb_ref, h1_ref):
    u = x_ref[...] + agg_ref[0, :_N, :] + agg_ref[1, :_N, :]
    z = jnp.dot(u, w1_ref[...], preferred_element_type=jnp.float32) + b1_ref[...]
    z = _bn_relu(z, g1_ref[...], be1_ref[...])
    z = jnp.dot(z, w2_ref[...], preferred_element_type=jnp.float32) + b2_ref[...]
    h1 = _bn_relu(z, bg_ref[...], bb_ref[...])
    h1_ref[...] = jnp.concatenate([h1, jnp.zeros((_N, _F - _H), jnp.float32)],
                                  axis=1)


def _dense1_body(x_ref, h1_ref, agg_ref, batch_ref, w1_ref, b1_ref, g1_ref,
                 be1_ref, w2_ref, b2_ref, bg_ref, bb_ref,
                 l0w_ref, l0b_ref, l1w_ref, l1b_ref, l2w_ref, l2b_ref,
                 hw_ref, out_ref):
    h1 = h1_ref[:, :_H]
    u = h1 + agg_ref[0, :_N, :_H] + agg_ref[1, :_N, :_H]
    z = jnp.dot(u, w1_ref[...], preferred_element_type=jnp.float32) + b1_ref[...]
    z = _bn_relu(z, g1_ref[...], be1_ref[...])
    z = jnp.dot(z, w2_ref[...], preferred_element_type=jnp.float32) + b2_ref[...]
    h2 = _bn_relu(z, bg_ref[...], bb_ref[...])

    iot = lax.broadcasted_iota(jnp.int32, (_B, _N), 0)
    oh = jnp.where(iot == batch_ref[...], 1.0, 0.0).astype(jnp.float32)
    cnt = jnp.sum(oh, axis=1, keepdims=True)
    scale = 1.0 / jnp.maximum(cnt, 1.0)
    p0 = jnp.dot(oh, x_ref[...], preferred_element_type=jnp.float32) * scale
    p1 = jnp.dot(oh, h1, preferred_element_type=jnp.float32) * scale
    p2 = jnp.dot(oh, h2, preferred_element_type=jnp.float32) * scale
    hw = hw_ref[...]
    o = (jnp.dot(p0, l0w_ref[...], preferred_element_type=jnp.float32)
         + l0b_ref[...]) * hw[:, 0:1]
    o += (jnp.dot(p1, l1w_ref[...], preferred_element_type=jnp.float32)
          + l1b_ref[...]) * hw[:, 1:2]
    o += (jnp.dot(p2, l2w_ref[...], preferred_element_type=jnp.float32)
          + l2b_ref[...]) * hw[:, 2:3]
    out_ref[...] = o


_dense0 = pl.pallas_call(
    _dense0_body,
    out_shape=jax.ShapeDtypeStruct((_N, _F), jnp.float32),
)

_dense1 = pl.pallas_call(
    _dense1_body,
    out_shape=jax.ShapeDtypeStruct((_B, _C), jnp.float32),
)


def kernel(x, edge_index, batch, hop_weights,
           c0_w1, c0_b1, c0_g1, c0_be1, c0_w2, c0_b2,
           c1_w1, c1_b1, c1_g1, c1_be1, c1_w2, c1_b2,
           bn0_g, bn0_b, bn1_g, bn1_b,
           l0_w, l0_b, l1_w, l1_b, l2_w, l2_b):
    pad = _ROW_PAD * _CH - _E
    src = jnp.concatenate([edge_index[0], jnp.zeros((pad,), jnp.int32)])
    dst = jnp.concatenate([edge_index[1], jnp.full((pad,), _N, jnp.int32)])
    src2 = src.reshape(-1, _CH)
    dst2 = dst.reshape(-1, _CH)
    zf = jnp.zeros((_RPT, _F), jnp.float32)

    r = lambda a: a.reshape(1, -1)

    agg0 = _sc_agg_f(src2, dst2, x, zf)
    h1 = _dense0(x, agg0, c0_w1, r(c0_b1), r(c0_g1), r(c0_be1),
                 c0_w2, r(c0_b2), r(bn0_g), r(bn0_b))
    agg1 = _sc_agg_f(src2, dst2, h1, zf)
    out = _dense1(x, h1, agg1, batch.reshape(1, _N), c1_w1, r(c1_b1),
                  r(c1_g1), r(c1_be1), c1_w2, r(c1_b2), r(bn1_g), r(bn1_b),
                  l0_w, r(l0_b), l1_w, r(l1_b), l2_w, r(l2_b), hop_weights)
    return out


# speedup vs baseline: 1.1942x; 1.0116x over previous
import functools

import jax
import jax.numpy as jnp
from jax import lax
from jax.experimental import pallas as pl
from jax.experimental.pallas import tpu as pltpu
from jax.experimental.pallas import tpu_sc as plsc

_N = 10000
_E = 320000
_F = 128
_H = 64
_B = 16
_C = 10

_NW = 32
_CH = 32
_CPT0 = 512
_CPT1 = 128
_NROW = 16 * (_CPT0 + _CPT1)
_E_PAD = _NROW * _CH
_PCH = 96
_NPASS = 6
_ROW_PAD = _NROW + 2 * _PCH
_N_PAD = 10112
_RPT = _N_PAD // 16


def _make_sc_agg(F):
    mesh = plsc.VectorSubcoreMesh(core_axis_name="c", subcore_axis_name="s")

    NB = 4

    @functools.partial(
        pl.kernel,
        out_type=jax.ShapeDtypeStruct((2, _N_PAD, F), jnp.float32),
        mesh=mesh,
        scratch_types=[
            pltpu.VMEM((_PCH, _CH), jnp.int32),
            pltpu.VMEM((_PCH, _CH), jnp.int32),
            pltpu.VMEM((_CH, F), jnp.float32),
            pltpu.VMEM((_CH, F), jnp.float32),
            pltpu.VMEM((_CH, F), jnp.float32),
            pltpu.VMEM((_CH, F), jnp.float32),
            pltpu.VMEM_SHARED((_N_PAD, F), jnp.float32),
            pltpu.SemaphoreType.DMA,
            pltpu.SemaphoreType.DMA,
        ],
    )
    def agg(src_hbm, dst_hbm, h_hbm, zrow_hbm, out_hbm,
            src_v, dst_v, rows0, rows1, rows2, rows3, acc, gsem, ssem):
        bufs = (rows0, rows1, rows2, rows3)
        cid = lax.axis_index("c")
        sid = lax.axis_index("s")
        pltpu.sync_copy(zrow_hbm, acc.at[pl.ds(sid * _RPT, _RPT)])
        plsc.subcore_barrier()

        my_base = jnp.where(cid == 0, sid * _CPT0, 16 * _CPT0 + sid * _CPT1)
        my_n = jnp.where(cid == 0, _CPT0, _CPT1)

        for p in range(_NPASS):
            cnt = jnp.minimum(my_n - p * _PCH, _PCH)

            @pl.when(cnt > 0)
            def _one_pass(p=p, cnt=cnt):
                pltpu.sync_copy(
                    src_hbm.at[pl.ds(my_base + p * _PCH, _PCH)], src_v)
                pltpu.sync_copy(
                    dst_hbm.at[pl.ds(my_base + p * _PCH, _PCH)], dst_v)
                pltpu.async_copy(h_hbm.at[src_v.at[0]], bufs[0], gsem)
                pltpu.async_copy(h_hbm.at[src_v.at[1]], bufs[1], gsem)
                pltpu.async_copy(h_hbm.at[src_v.at[2]], bufs[2], gsem)

                def body(t, carry):
                    r = lax.rem(t, NB)
                    pltpu.make_async_copy(h_hbm.at[src_v.at[t]], bufs[0],
                                          gsem).wait()
                    for i in range(NB):

                        @pl.when(r == i)
                        def _scatter(i=i):
                            pltpu.async_copy(bufs[i], acc.at[dst_v.at[t]],
                                             ssem, add=True)

                    @pl.when(t + 3 < cnt)
                    def _prefetch():
                        @pl.when(t >= 1)
                        def _drain():
                            pltpu.make_async_copy(h_hbm.at[src_v.at[t]],
                                                  bufs[0], ssem).wait()

                        r2 = lax.rem(t + 3, NB)
                        for i in range(NB):

                            @pl.when(r2 == i)
                            def _gather(i=i):
                                pltpu.async_copy(h_hbm.at[src_v.at[t + 3]],
                                                 bufs[i], gsem)

                    return carry

                lax.fori_loop(0, cnt, body, 0)
                for _ in range(4):
                    pltpu.make_async_copy(h_hbm.at[src_v.at[0]], bufs[0],
                                          ssem).wait()

        plsc.subcore_barrier()
        pltpu.sync_copy(acc.at[pl.ds(sid * _RPT, _RPT)],
                        out_hbm.at[cid, pl.ds(sid * _RPT, _RPT)])

    return agg


_sc_agg_f = _make_sc_agg(_F)


def _bn_relu(z, g, b):
    m = jnp.mean(z, axis=0, keepdims=True)
    v = jnp.mean((z - m) * (z - m), axis=0, keepdims=True)
    return jax.nn.relu((z - m) * lax.rsqrt(v + 1e-5) * g + b)


def _dense0_body(x_ref, agg_ref, w1_ref, b1_ref, g1_ref, be1_ref,
                 w2_ref, b2_ref, bg_ref, bb_ref, h1_ref):
    u = x_ref[...] + agg_ref[0, :_N, :] + agg_ref[1, :_N, :]
    z = jnp.dot(u, w1_ref[...], preferred_element_type=jnp.float32) + b1_ref[...]
    z = _bn_relu(z, g1_ref[...], be1_ref[...])
    z = jnp.dot(z, w2_ref[...], preferred_element_type=jnp.float32) + b2_ref[...]
    h1 = _bn_relu(z, bg_ref[...], bb_ref[...])
    h1_ref[...] = jnp.concatenate([h1, jnp.zeros((_N, _F - _H), jnp.float32)],
                                  axis=1)


def _dense1_body(x_ref, h1_ref, agg_ref, batch_ref, w1_ref, b1_ref, g1_ref,
                 be1_ref, w2_ref, b2_ref, bg_ref, bb_ref,
                 l0w_ref, l0b_ref, l1w_ref, l1b_ref, l2w_ref, l2b_ref,
                 hw_ref, out_ref):
    h1 = h1_ref[:, :_H]
    u = h1 + agg_ref[0, :_N, :_H] + agg_ref[1, :_N, :_H]
    z = jnp.dot(u, w1_ref[...], preferred_element_type=jnp.float32) + b1_ref[...]
    z = _bn_relu(z, g1_ref[...], be1_ref[...])
    z = jnp.dot(z, w2_ref[...], preferred_element_type=jnp.float32) + b2_ref[...]
    h2 = _bn_relu(z, bg_ref[...], bb_ref[...])

    iot = lax.broadcasted_iota(jnp.int32, (_B, _N), 0)
    oh = jnp.where(iot == batch_ref[...], 1.0, 0.0).astype(jnp.float32)
    cnt = jnp.sum(oh, axis=1, keepdims=True)
    scale = 1.0 / jnp.maximum(cnt, 1.0)
    p0 = jnp.dot(oh, x_ref[...], preferred_element_type=jnp.float32) * scale
    p1 = jnp.dot(oh, h1, preferred_element_type=jnp.float32) * scale
    p2 = jnp.dot(oh, h2, preferred_element_type=jnp.float32) * scale
    hw = hw_ref[...]
    o = (jnp.dot(p0, l0w_ref[...], preferred_element_type=jnp.float32)
         + l0b_ref[...]) * hw[:, 0:1]
    o += (jnp.dot(p1, l1w_ref[...], preferred_element_type=jnp.float32)
          + l1b_ref[...]) * hw[:, 1:2]
    o += (jnp.dot(p2, l2w_ref[...], preferred_element_type=jnp.float32)
          + l2b_ref[...]) * hw[:, 2:3]
    out_ref[...] = o


_dense0 = pl.pallas_call(
    _dense0_body,
    out_shape=jax.ShapeDtypeStruct((_N, _F), jnp.float32),
)

_dense1 = pl.pallas_call(
    _dense1_body,
    out_shape=jax.ShapeDtypeStruct((_B, _C), jnp.float32),
)


def kernel(x, edge_index, batch, hop_weights,
           c0_w1, c0_b1, c0_g1, c0_be1, c0_w2, c0_b2,
           c1_w1, c1_b1, c1_g1, c1_be1, c1_w2, c1_b2,
           bn0_g, bn0_b, bn1_g, bn1_b,
           l0_w, l0_b, l1_w, l1_b, l2_w, l2_b):
    pad = _ROW_PAD * _CH - _E
    src = jnp.concatenate([edge_index[0], jnp.zeros((pad,), jnp.int32)])
    dst = jnp.concatenate([edge_index[1], jnp.full((pad,), _N, jnp.int32)])
    src2 = src.reshape(-1, _CH)
    dst2 = dst.reshape(-1, _CH)
    zf = jnp.zeros((_RPT, _F), jnp.float32)

    r = lambda a: a.reshape(1, -1)

    agg0 = _sc_agg_f(src2, dst2, x, zf)
    h1 = _dense0(x, agg0, c0_w1, r(c0_b1), r(c0_g1), r(c0_be1),
                 c0_w2, r(c0_b2), r(bn0_g), r(bn0_b))
    agg1 = _sc_agg_f(src2, dst2, h1, zf)
    out = _dense1(x, h1, agg1, batch.reshape(1, _N), c1_w1, r(c1_b1),
                  r(c1_g1), r(c1_be1), c1_w2, r(c1_b2), r(bn1_g), r(bn1_b),
                  l0_w, r(l0_b), l1_w, r(l1_b), l2_w, r(l2_b), hop_weights)
    return out


# speedup vs baseline: 1.2423x; 1.0403x over previous
import functools

import jax
import jax.numpy as jnp
from jax import lax
from jax.experimental import pallas as pl
from jax.experimental.pallas import tpu as pltpu
from jax.experimental.pallas import tpu_sc as plsc

_N = 10000
_E = 320000
_F = 128
_H = 64
_B = 16
_C = 10

_NW = 32
_CH = 32
_CPT0 = 560
_CPT1 = 80
_NROW = 16 * (_CPT0 + _CPT1)
_E_PAD = _NROW * _CH
_PCH = 96
_NPASS = 6
_ROW_PAD = _NROW + 2 * _PCH
_N_PAD = 10112
_RPT = _N_PAD // 16


def _make_sc_agg(F):
    mesh = plsc.VectorSubcoreMesh(core_axis_name="c", subcore_axis_name="s")

    NB = 4

    @functools.partial(
        pl.kernel,
        out_type=jax.ShapeDtypeStruct((2, _N_PAD, F), jnp.float32),
        mesh=mesh,
        scratch_types=[
            pltpu.VMEM((_PCH, _CH), jnp.int32),
            pltpu.VMEM((_PCH, _CH), jnp.int32),
            pltpu.VMEM((_CH, F), jnp.float32),
            pltpu.VMEM((_CH, F), jnp.float32),
            pltpu.VMEM((_CH, F), jnp.float32),
            pltpu.VMEM((_CH, F), jnp.float32),
            pltpu.VMEM_SHARED((_N_PAD, F), jnp.float32),
            pltpu.SemaphoreType.DMA,
            pltpu.SemaphoreType.DMA,
        ],
    )
    def agg(src_hbm, dst_hbm, h_hbm, zrow_hbm, out_hbm,
            src_v, dst_v, rows0, rows1, rows2, rows3, acc, gsem, ssem):
        bufs = (rows0, rows1, rows2, rows3)
        cid = lax.axis_index("c")
        sid = lax.axis_index("s")
        pltpu.sync_copy(zrow_hbm, acc.at[pl.ds(sid * _RPT, _RPT)])
        plsc.subcore_barrier()

        my_base = jnp.where(cid == 0, sid * _CPT0, 16 * _CPT0 + sid * _CPT1)
        my_n = jnp.where(cid == 0, _CPT0, _CPT1)

        for p in range(_NPASS):
            cnt = jnp.minimum(my_n - p * _PCH, _PCH)

            @pl.when(cnt > 0)
            def _one_pass(p=p, cnt=cnt):
                pltpu.sync_copy(
                    src_hbm.at[pl.ds(my_base + p * _PCH, _PCH)], src_v)
                pltpu.sync_copy(
                    dst_hbm.at[pl.ds(my_base + p * _PCH, _PCH)], dst_v)
                pltpu.async_copy(h_hbm.at[src_v.at[0]], bufs[0], gsem)
                pltpu.async_copy(h_hbm.at[src_v.at[1]], bufs[1], gsem)
                pltpu.async_copy(h_hbm.at[src_v.at[2]], bufs[2], gsem)

                def body(t, carry):
                    r = lax.rem(t, NB)
                    pltpu.make_async_copy(h_hbm.at[src_v.at[t]], bufs[0],
                                          gsem).wait()
                    for i in range(NB):

                        @pl.when(r == i)
                        def _scatter(i=i):
                            pltpu.async_copy(bufs[i], acc.at[dst_v.at[t]],
                                             ssem, add=True)

                    @pl.when(t + 3 < cnt)
                    def _prefetch():
                        @pl.when(t >= 1)
                        def _drain():
                            pltpu.make_async_copy(h_hbm.at[src_v.at[t]],
                                                  bufs[0], ssem).wait()

                        r2 = lax.rem(t + 3, NB)
                        for i in range(NB):

                            @pl.when(r2 == i)
                            def _gather(i=i):
                                pltpu.async_copy(h_hbm.at[src_v.at[t + 3]],
                                                 bufs[i], gsem)

                    return carry

                lax.fori_loop(0, cnt, body, 0)
                for _ in range(4):
                    pltpu.make_async_copy(h_hbm.at[src_v.at[0]], bufs[0],
                                          ssem).wait()

        plsc.subcore_barrier()
        pltpu.sync_copy(acc.at[pl.ds(sid * _RPT, _RPT)],
                        out_hbm.at[cid, pl.ds(sid * _RPT, _RPT)])

    return agg


_sc_agg_f = _make_sc_agg(_F)


def _bn_relu(z, g, b):
    m = jnp.mean(z, axis=0, keepdims=True)
    v = jnp.mean((z - m) * (z - m), axis=0, keepdims=True)
    return jax.nn.relu((z - m) * lax.rsqrt(v + 1e-5) * g + b)


def _dense0_body(x_ref, agg_ref, w1_ref, b1_ref, g1_ref, be1_ref,
                 w2_ref, b2_ref, bg_ref, bb_ref, h1_ref):
    u = x_ref[...] + agg_ref[0, :_N, :] + agg_ref[1, :_N, :]
    z = jnp.dot(u, w1_ref[...], preferred_element_type=jnp.float32) + b1_ref[...]
    z = _bn_relu(z, g1_ref[...], be1_ref[...])
    z = jnp.dot(z, w2_ref[...], preferred_element_type=jnp.float32) + b2_ref[...]
    h1 = _bn_relu(z, bg_ref[...], bb_ref[...])
    h1_ref[...] = jnp.concatenate([h1, jnp.zeros((_N, _F - _H), jnp.float32)],
                                  axis=1)


def _dense1_body(x_ref, h1_ref, agg_ref, batch_ref, w1_ref, b1_ref, g1_ref,
                 be1_ref, w2_ref, b2_ref, bg_ref, bb_ref,
                 l0w_ref, l0b_ref, l1w_ref, l1b_ref, l2w_ref, l2b_ref,
                 hw_ref, out_ref):
    h1 = h1_ref[:, :_H]
    u = h1 + agg_ref[0, :_N, :_H] + agg_ref[1, :_N, :_H]
    z = jnp.dot(u, w1_ref[...], preferred_element_type=jnp.float32) + b1_ref[...]
    z = _bn_relu(z, g1_ref[...], be1_ref[...])
    z = jnp.dot(z, w2_ref[...], preferred_element_type=jnp.float32) + b2_ref[...]
    h2 = _bn_relu(z, bg_ref[...], bb_ref[...])

    iot = lax.broadcasted_iota(jnp.int32, (_B, _N), 0)
    oh = jnp.where(iot == batch_ref[...], 1.0, 0.0).astype(jnp.float32)
    cnt = jnp.sum(oh, axis=1, keepdims=True)
    scale = 1.0 / jnp.maximum(cnt, 1.0)
    p0 = jnp.dot(oh, x_ref[...], preferred_element_type=jnp.float32) * scale
    p1 = jnp.dot(oh, h1, preferred_element_type=jnp.float32) * scale
    p2 = jnp.dot(oh, h2, preferred_element_type=jnp.float32) * scale
    hw = hw_ref[...]
    o = (jnp.dot(p0, l0w_ref[...], preferred_element_type=jnp.float32)
         + l0b_ref[...]) * hw[:, 0:1]
    o += (jnp.dot(p1, l1w_ref[...], preferred_element_type=jnp.float32)
          + l1b_ref[...]) * hw[:, 1:2]
    o += (jnp.dot(p2, l2w_ref[...], preferred_element_type=jnp.float32)
          + l2b_ref[...]) * hw[:, 2:3]
    out_ref[...] = o


_dense0 = pl.pallas_call(
    _dense0_body,
    out_shape=jax.ShapeDtypeStruct((_N, _F), jnp.float32),
)

_dense1 = pl.pallas_call(
    _dense1_body,
    out_shape=jax.ShapeDtypeStruct((_B, _C), jnp.float32),
)


def kernel(x, edge_index, batch, hop_weights,
           c0_w1, c0_b1, c0_g1, c0_be1, c0_w2, c0_b2,
           c1_w1, c1_b1, c1_g1, c1_be1, c1_w2, c1_b2,
           bn0_g, bn0_b, bn1_g, bn1_b,
           l0_w, l0_b, l1_w, l1_b, l2_w, l2_b):
    pad = _ROW_PAD * _CH - _E
    src = jnp.concatenate([edge_index[0], jnp.zeros((pad,), jnp.int32)])
    dst = jnp.concatenate([edge_index[1], jnp.full((pad,), _N, jnp.int32)])
    src2 = src.reshape(-1, _CH)
    dst2 = dst.reshape(-1, _CH)
    zf = jnp.zeros((_RPT, _F), jnp.float32)

    r = lambda a: a.reshape(1, -1)

    agg0 = _sc_agg_f(src2, dst2, x, zf)
    h1 = _dense0(x, agg0, c0_w1, r(c0_b1), r(c0_g1), r(c0_be1),
                 c0_w2, r(c0_b2), r(bn0_g), r(bn0_b))
    agg1 = _sc_agg_f(src2, dst2, h1, zf)
    out = _dense1(x, h1, agg1, batch.reshape(1, _N), c1_w1, r(c1_b1),
                  r(c1_g1), r(c1_be1), c1_w2, r(c1_b2), r(bn1_g), r(bn1_b),
                  l0_w, r(l0_b), l1_w, r(l1_b), l2_w, r(l2_b), hop_weights)
    return out


# speedup vs baseline: 1.3333x; 1.0733x over previous
import functools

import jax
import jax.numpy as jnp
from jax import lax
from jax.experimental import pallas as pl
from jax.experimental.pallas import tpu as pltpu
from jax.experimental.pallas import tpu_sc as plsc

_N = 10000
_E = 320000
_F = 128
_H = 64
_B = 16
_C = 10

_NW = 32
_CH = 32
_CPT0 = 592
_CPT1 = 48
_NROW = 16 * (_CPT0 + _CPT1)
_E_PAD = _NROW * _CH
_PCH = 96
_NPASS = 7
_ROW_PAD = _NROW + 2 * _PCH
_N_PAD = 10112
_RPT = _N_PAD // 16


def _make_sc_agg(F):
    mesh = plsc.VectorSubcoreMesh(core_axis_name="c", subcore_axis_name="s")

    NB = 4

    @functools.partial(
        pl.kernel,
        out_type=jax.ShapeDtypeStruct((2, _N_PAD, F), jnp.float32),
        mesh=mesh,
        scratch_types=[
            pltpu.VMEM((_PCH, _CH), jnp.int32),
            pltpu.VMEM((_PCH, _CH), jnp.int32),
            pltpu.VMEM((_CH, F), jnp.float32),
            pltpu.VMEM((_CH, F), jnp.float32),
            pltpu.VMEM((_CH, F), jnp.float32),
            pltpu.VMEM((_CH, F), jnp.float32),
            pltpu.VMEM_SHARED((_N_PAD, F), jnp.float32),
            pltpu.SemaphoreType.DMA,
            pltpu.SemaphoreType.DMA,
        ],
    )
    def agg(src_hbm, dst_hbm, h_hbm, zrow_hbm, out_hbm,
            src_v, dst_v, rows0, rows1, rows2, rows3, acc, gsem, ssem):
        bufs = (rows0, rows1, rows2, rows3)
        cid = lax.axis_index("c")
        sid = lax.axis_index("s")
        pltpu.sync_copy(zrow_hbm, acc.at[pl.ds(sid * _RPT, _RPT)])
        plsc.subcore_barrier()

        my_base = jnp.where(cid == 0, sid * _CPT0, 16 * _CPT0 + sid * _CPT1)
        my_n = jnp.where(cid == 0, _CPT0, _CPT1)

        for p in range(_NPASS):
            cnt = jnp.minimum(my_n - p * _PCH, _PCH)

            @pl.when(cnt > 0)
            def _one_pass(p=p, cnt=cnt):
                pltpu.sync_copy(
                    src_hbm.at[pl.ds(my_base + p * _PCH, _PCH)], src_v)
                pltpu.sync_copy(
                    dst_hbm.at[pl.ds(my_base + p * _PCH, _PCH)], dst_v)
                pltpu.async_copy(h_hbm.at[src_v.at[0]], bufs[0], gsem)
                pltpu.async_copy(h_hbm.at[src_v.at[1]], bufs[1], gsem)
                pltpu.async_copy(h_hbm.at[src_v.at[2]], bufs[2], gsem)

                def body(t, carry):
                    r = lax.rem(t, NB)
                    pltpu.make_async_copy(h_hbm.at[src_v.at[t]], bufs[0],
                                          gsem).wait()
                    for i in range(NB):

                        @pl.when(r == i)
                        def _scatter(i=i):
                            pltpu.async_copy(bufs[i], acc.at[dst_v.at[t]],
                                             ssem, add=True)

                    @pl.when(t + 3 < cnt)
                    def _prefetch():
                        @pl.when(t >= 1)
                        def _drain():
                            pltpu.make_async_copy(h_hbm.at[src_v.at[t]],
                                                  bufs[0], ssem).wait()

                        r2 = lax.rem(t + 3, NB)
                        for i in range(NB):

                            @pl.when(r2 == i)
                            def _gather(i=i):
                                pltpu.async_copy(h_hbm.at[src_v.at[t + 3]],
                                                 bufs[i], gsem)

                    return carry

                lax.fori_loop(0, cnt, body, 0)
                for _ in range(4):
                    pltpu.make_async_copy(h_hbm.at[src_v.at[0]], bufs[0],
                                          ssem).wait()

        plsc.subcore_barrier()
        pltpu.sync_copy(acc.at[pl.ds(sid * _RPT, _RPT)],
                        out_hbm.at[cid, pl.ds(sid * _RPT, _RPT)])

    return agg


_sc_agg_f = _make_sc_agg(_F)


def _bn_relu(z, g, b):
    m = jnp.mean(z, axis=0, keepdims=True)
    v = jnp.mean((z - m) * (z - m), axis=0, keepdims=True)
    return jax.nn.relu((z - m) * lax.rsqrt(v + 1e-5) * g + b)


def _dense0_body(x_ref, agg_ref, w1_ref, b1_ref, g1_ref, be1_ref,
                 w2_ref, b2_ref, bg_ref, bb_ref, h1_ref):
    u = x_ref[...] + agg_ref[0, :_N, :] + agg_ref[1, :_N, :]
    z = jnp.dot(u, w1_ref[...], preferred_element_type=jnp.float32) + b1_ref[...]
    z = _bn_relu(z, g1_ref[...], be1_ref[...])
    z = jnp.dot(z, w2_ref[...], preferred_element_type=jnp.float32) + b2_ref[...]
    h1 = _bn_relu(z, bg_ref[...], bb_ref[...])
    h1_ref[...] = jnp.concatenate([h1, jnp.zeros((_N, _F - _H), jnp.float32)],
                                  axis=1)


def _dense1_body(x_ref, h1_ref, agg_ref, batch_ref, w1_ref, b1_ref, g1_ref,
                 be1_ref, w2_ref, b2_ref, bg_ref, bb_ref,
                 l0w_ref, l0b_ref, l1w_ref, l1b_ref, l2w_ref, l2b_ref,
                 hw_ref, out_ref):
    h1 = h1_ref[:, :_H]
    u = h1 + agg_ref[0, :_N, :_H] + agg_ref[1, :_N, :_H]
    z = jnp.dot(u, w1_ref[...], preferred_element_type=jnp.float32) + b1_ref[...]
    z = _bn_relu(z, g1_ref[...], be1_ref[...])
    z = jnp.dot(z, w2_ref[...], preferred_element_type=jnp.float32) + b2_ref[...]
    h2 = _bn_relu(z, bg_ref[...], bb_ref[...])

    iot = lax.broadcasted_iota(jnp.int32, (_B, _N), 0)
    oh = jnp.where(iot == batch_ref[...], 1.0, 0.0).astype(jnp.float32)
    cnt = jnp.sum(oh, axis=1, keepdims=True)
    scale = 1.0 / jnp.maximum(cnt, 1.0)
    p0 = jnp.dot(oh, x_ref[...], preferred_element_type=jnp.float32) * scale
    p1 = jnp.dot(oh, h1, preferred_element_type=jnp.float32) * scale
    p2 = jnp.dot(oh, h2, preferred_element_type=jnp.float32) * scale
    hw = hw_ref[...]
    o = (jnp.dot(p0, l0w_ref[...], preferred_element_type=jnp.float32)
         + l0b_ref[...]) * hw[:, 0:1]
    o += (jnp.dot(p1, l1w_ref[...], preferred_element_type=jnp.float32)
          + l1b_ref[...]) * hw[:, 1:2]
    o += (jnp.dot(p2, l2w_ref[...], preferred_element_type=jnp.float32)
          + l2b_ref[...]) * hw[:, 2:3]
    out_ref[...] = o


_dense0 = pl.pallas_call(
    _dense0_body,
    out_shape=jax.ShapeDtypeStruct((_N, _F), jnp.float32),
)

_dense1 = pl.pallas_call(
    _dense1_body,
    out_shape=jax.ShapeDtypeStruct((_B, _C), jnp.float32),
)


def kernel(x, edge_index, batch, hop_weights,
           c0_w1, c0_b1, c0_g1, c0_be1, c0_w2, c0_b2,
           c1_w1, c1_b1, c1_g1, c1_be1, c1_w2, c1_b2,
           bn0_g, bn0_b, bn1_g, bn1_b,
           l0_w, l0_b, l1_w, l1_b, l2_w, l2_b):
    pad = _ROW_PAD * _CH - _E
    src = jnp.concatenate([edge_index[0], jnp.zeros((pad,), jnp.int32)])
    dst = jnp.concatenate([edge_index[1], jnp.full((pad,), _N, jnp.int32)])
    src2 = src.reshape(-1, _CH)
    dst2 = dst.reshape(-1, _CH)
    zf = jnp.zeros((_RPT, _F), jnp.float32)

    r = lambda a: a.reshape(1, -1)

    agg0 = _sc_agg_f(src2, dst2, x, zf)
    h1 = _dense0(x, agg0, c0_w1, r(c0_b1), r(c0_g1), r(c0_be1),
                 c0_w2, r(c0_b2), r(bn0_g), r(bn0_b))
    agg1 = _sc_agg_f(src2, dst2, h1, zf)
    out = _dense1(x, h1, agg1, batch.reshape(1, _N), c1_w1, r(c1_b1),
                  r(c1_g1), r(c1_be1), c1_w2, r(c1_b2), r(bn1_g), r(bn1_b),
                  l0_w, r(l0_b), l1_w, r(l1_b), l2_w, r(l2_b), hop_weights)
    return out
